# baseline (speedup 1.0000x reference)
import functools

import jax
import jax.numpy as jnp
from jax import lax
from jax.experimental import pallas as pl
from jax.experimental.pallas import tpu as pltpu
from jax.experimental.pallas import tpu_sc as plsc

_NC = 2
_NS = 16
_L = 16
_NW = _NC * _NS

_BS, _N, _D = 16, 128, 64
_ROWS = _BS * _N
_RPW = _ROWS // _NW
_T = 8


def _sc_body(msg_hbm, mm_hbm, out_hbm, t_v, mm_v, accs_v, idx_v, sem):
    wid = lax.axis_index("s") * _NC + lax.axis_index("c")
    base = wid * _RPW
    b = base >> 7
    i0 = pl.multiple_of(base & (_N - 1), _RPW)

    stride = _N * _D + 1

    step = lax.iota(jnp.int32, _L) * stride

    def idx_body(q, carry):
        c0 = (base * _D + q) * _N + i0
        for c in range(_RPW // _L):
            idx_v[q, pl.ds(c * _L, _L)] = c0 + c * _L * stride + step
        return carry

    lax.fori_loop(0, _D, idx_body, 0)

    def fire_body(q, carry):
        pltpu.make_async_copy(
            msg_hbm.at[idx_v.at[q]], t_v.at[q // _T, q % _T], sem
        ).start()
        return carry

    lax.fori_loop(0, _D, fire_body, 0)

    pltpu.sync_copy(mm_hbm.at[b, pl.ds(i0 // _T, _T)], mm_v)

    def row_body(k, carry):
        acc = mm_v[k // _T, k % _T, pl.ds(0, _L)]
        for c in range(1, _N // _L):
            acc = acc + mm_v[k // _T, k % _T, pl.ds(c * _L, _L)]
        accs_v[k, :] = acc
        return carry

    lax.fori_loop(0, _RPW, row_body, 0)

    rows16 = lax.iota(jnp.int32, _L)
    masks = []
    for c in range(_RPW // _L):
        ridx = c * _L + rows16
        s = plsc.load_gather(accs_v, [ridx, jnp.zeros((_L,), jnp.int32)])
        for l in range(1, _L):
            s = s + plsc.load_gather(
                accs_v, [ridx, jnp.full((_L,), l, jnp.int32)]
            )
        masks.append(
            jnp.where(s != 0.0, jnp.float32(1.0), jnp.float32(0.0))
        )

    def drain_body(q, carry):
        pltpu.make_async_copy(
            msg_hbm.at[idx_v.at[q]], t_v.at[q // _T, q % _T], sem
        ).wait()
        return carry

    lax.fori_loop(0, _D, drain_body, 0)

    def scale_body(q, carry):
        for c in range(_RPW // _L):
            t_v[q // _T, q % _T, pl.ds(c * _L, _L)] = (
                t_v[q // _T, q % _T, pl.ds(c * _L, _L)] * masks[c]
            )
        return carry

    lax.fori_loop(0, _D, scale_body, 0)

    pltpu.sync_copy(t_v, out_hbm.at[b, :, :, pl.ds(i0, _RPW)])


@functools.cache
def _sc_call():
    return pl.kernel(
        _sc_body,
        out_type=jax.ShapeDtypeStruct((_BS, _T, _T, _N), jnp.float32),
        mesh=plsc.VectorSubcoreMesh(
            core_axis_name="c", subcore_axis_name="s",
            num_cores=_NC, num_subcores=_NS,
        ),
        scratch_types=[
            pltpu.VMEM((_T, _T, _RPW), jnp.float32),
            pltpu.VMEM((_T, _T, _N), jnp.float32),
            pltpu.VMEM((_RPW, _L), jnp.float32),
            pltpu.VMEM((_D, _RPW), jnp.int32),
            pltpu.SemaphoreType.DMA,
        ],
        compiler_params=pltpu.CompilerParams(
            use_tc_tiling_on_sc=False,
            needs_layout_passes=False,
            disable_bounds_checks=True,
        ),
    )


@jax.jit
def kernel(msg, msg_matrix):
    bs, n, _, d = msg.shape
    msg_flat = jnp.transpose(msg, (0, 1, 3, 2)).reshape(-1)
    mm4 = msg_matrix.reshape(bs, n // _T, _T, n)
    out5 = _sc_call()(msg_flat, mm4)
    return jnp.transpose(out5.reshape(bs, d, n), (0, 2, 1))


# speedup vs baseline: 1.0039x; 1.0039x over previous
import functools

import jax
import jax.numpy as jnp
from jax import lax
from jax.experimental import pallas as pl
from jax.experimental.pallas import tpu as pltpu
from jax.experimental.pallas import tpu_sc as plsc

_NC = 2
_NS = 16
_L = 16
_NW = _NC * _NS

_BS, _N, _D = 16, 128, 64
_ROWS = _BS * _N
_RPW = _ROWS // _NW
_T = 8


def _sc_body(msg_hbm, mm_hbm, out_hbm, t_v, mm_v, accs_v, idx_v, sem):
    wid = lax.axis_index("s") * _NC + lax.axis_index("c")
    base = wid * _RPW
    b = base >> 7
    i0 = pl.multiple_of(base & (_N - 1), _RPW)

    stride = _N * _D + 1

    step = lax.iota(jnp.int32, _L) * stride

    def idx_body(q, carry):
        c0 = (base * _D + q) * _N + i0
        for c in range(_RPW // _L):
            idx_v[q, pl.ds(c * _L, _L)] = c0 + c * _L * stride + step
        return carry

    lax.fori_loop(0, _D, idx_body, 0)

    def fire_body(q, carry):
        pltpu.make_async_copy(
            msg_hbm.at[idx_v.at[q]], t_v.at[q // _T, q % _T], sem
        ).start()
        return carry

    lax.fori_loop(0, _D, fire_body, 0)

    pltpu.sync_copy(mm_hbm.at[b, pl.ds(i0 // _T, _T)], mm_v)

    def row_body(k, carry):
        acc = mm_v[k // _T, k % _T, pl.ds(0, _L)]
        for c in range(1, _N // _L):
            acc = acc + mm_v[k // _T, k % _T, pl.ds(c * _L, _L)]
        accs_v[k, :] = acc
        return carry

    lax.fori_loop(0, _RPW, row_body, 0)

    rows16 = lax.iota(jnp.int32, _L)
    masks = []
    for c in range(_RPW // _L):
        ridx = c * _L + rows16
        s = plsc.load_gather(accs_v, [ridx, jnp.zeros((_L,), jnp.int32)])
        for l in range(1, _L):
            s = s + plsc.load_gather(
                accs_v, [ridx, jnp.full((_L,), l, jnp.int32)]
            )
        masks.append(
            jnp.where(s != 0.0, jnp.float32(1.0), jnp.float32(0.0))
        )

    def drain_body(q, carry):
        pltpu.make_async_copy(
            msg_hbm.at[idx_v.at[q]], t_v.at[q // _T, q % _T], sem
        ).wait()
        return carry

    lax.fori_loop(0, _D, drain_body, 0)

    def scale_body(q, carry):
        for c in range(_RPW // _L):
            t_v[q // _T, q % _T, pl.ds(c * _L, _L)] = (
                t_v[q // _T, q % _T, pl.ds(c * _L, _L)] * masks[c]
            )
        return carry

    lax.fori_loop(0, _D, scale_body, 0)

    pltpu.sync_copy(t_v, out_hbm.at[b, :, :, pl.ds(i0, _RPW)])


@functools.cache
def _sc_call():
    return pl.kernel(
        _sc_body,
        out_type=jax.ShapeDtypeStruct((_BS, _T, _T, _N), jnp.float32),
        mesh=plsc.VectorSubcoreMesh(
            core_axis_name="c", subcore_axis_name="s",
            num_cores=_NC, num_subcores=_NS,
        ),
        scratch_types=[
            pltpu.VMEM((_T, _T, _RPW), jnp.float32),
            pltpu.VMEM((_T, _T, _N), jnp.float32),
            pltpu.VMEM((_RPW, _L), jnp.float32),
            pltpu.VMEM((_D, _RPW), jnp.int32),
            pltpu.SemaphoreType.DMA,
        ],
        compiler_params=pltpu.CompilerParams(
            use_tc_tiling_on_sc=False,
            needs_layout_passes=False,
            disable_bounds_checks=True,
            skip_device_barrier=True,
        ),
    )


@jax.jit
def kernel(msg, msg_matrix):
    bs, n, _, d = msg.shape
    msg_flat = jnp.transpose(msg, (0, 1, 3, 2)).reshape(-1)
    mm4 = msg_matrix.reshape(bs, n // _T, _T, n)
    out5 = _sc_call()(msg_flat, mm4)
    return jnp.transpose(out5.reshape(bs, d, n), (0, 2, 1))


# speedup vs baseline: 1.0242x; 1.0203x over previous
import functools

import jax
import jax.numpy as jnp
from jax import lax
from jax.experimental import pallas as pl
from jax.experimental.pallas import tpu as pltpu
from jax.experimental.pallas import tpu_sc as plsc

_NC = 2
_NS = 16
_L = 16
_NW = _NC * _NS

_BS, _N, _D = 16, 128, 64
_ROWS = _BS * _N
_RPW = _ROWS // _NW
_T = 8


def _sc_body(msg_hbm, mm_hbm, out_hbm, t_v, mm_v, accs_v, idx_v, sem):
    wid = lax.axis_index("s") * _NC + lax.axis_index("c")
    base = wid * _RPW
    b = base >> 7
    i0 = pl.multiple_of(base & (_N - 1), _RPW)

    stride = _N * _D + 1

    step = lax.iota(jnp.int32, _L) * stride

    def fire_body(g, carry):
        for sub in range(_T):
            c0 = (base * _D + g * _T + sub) * _N + i0
            for c in range(_RPW // _L):
                idx_v[g, pl.ds(sub * _RPW + c * _L, _L)] = (
                    c0 + c * _L * stride + step
                )
        for sub in range(_T):
            pltpu.make_async_copy(
                msg_hbm.at[idx_v.at[g, pl.ds(sub * _RPW, _RPW)]],
                t_v.at[g, sub],
                sem,
            ).start()
        return carry

    lax.fori_loop(0, _T, fire_body, 0)

    pltpu.sync_copy(mm_hbm.at[b, pl.ds(i0 // _T, _T)], mm_v)

    def row_body(k, carry):
        acc = mm_v[k // _T, k % _T, pl.ds(0, _L)]
        for c in range(1, _N // _L):
            acc = acc + mm_v[k // _T, k % _T, pl.ds(c * _L, _L)]
        accs_v[k, :] = acc
        return carry

    lax.fori_loop(0, _RPW, row_body, 0)

    rows16 = lax.iota(jnp.int32, _L)
    masks = []
    for c in range(_RPW // _L):
        ridx = c * _L + rows16
        s = plsc.load_gather(accs_v, [ridx, jnp.zeros((_L,), jnp.int32)])
        for l in range(1, _L):
            s = s + plsc.load_gather(
                accs_v, [ridx, jnp.full((_L,), l, jnp.int32)]
            )
        masks.append(
            jnp.where(s != 0.0, jnp.float32(1.0), jnp.float32(0.0))
        )

    def drain_body(g, carry):
        for sub in range(_T):
            pltpu.make_async_copy(
                msg_hbm.at[idx_v.at[g, pl.ds(sub * _RPW, _RPW)]],
                t_v.at[g, sub],
                sem,
            ).wait()
        for sub in range(_T):
            for c in range(_RPW // _L):
                t_v[g, sub, pl.ds(c * _L, _L)] = (
                    t_v[g, sub, pl.ds(c * _L, _L)] * masks[c]
                )
        return carry

    lax.fori_loop(0, _T, drain_body, 0)

    pltpu.sync_copy(t_v, out_hbm.at[b, :, :, pl.ds(i0, _RPW)])


@functools.cache
def _sc_call():
    return pl.kernel(
        _sc_body,
        out_type=jax.ShapeDtypeStruct((_BS, _T, _T, _N), jnp.float32),
        mesh=plsc.VectorSubcoreMesh(
            core_axis_name="c", subcore_axis_name="s",
            num_cores=_NC, num_subcores=_NS,
        ),
        scratch_types=[
            pltpu.VMEM((_T, _T, _RPW), jnp.float32),
            pltpu.VMEM((_T, _T, _N), jnp.float32),
            pltpu.VMEM((_RPW, _L), jnp.float32),
            pltpu.VMEM((_T, _T * _RPW), jnp.int32),
            pltpu.SemaphoreType.DMA,
        ],
        compiler_params=pltpu.CompilerParams(
            use_tc_tiling_on_sc=False,
            needs_layout_passes=False,
            disable_bounds_checks=True,
            skip_device_barrier=True,
        ),
    )


@jax.jit
def kernel(msg, msg_matrix):
    bs, n, _, d = msg.shape
    msg_flat = jnp.transpose(msg, (0, 1, 3, 2)).reshape(-1)
    mm4 = msg_matrix.reshape(bs, n // _T, _T, n)
    out5 = _sc_call()(msg_flat, mm4)
    return jnp.transpose(out5.reshape(bs, d, n), (0, 2, 1))
